# baseline (speedup 1.0000x reference)
import functools

import jax
import jax.numpy as jnp
from jax import lax
from jax.experimental import pallas as pl
from jax.experimental.pallas import tpu as pltpu
from jax.experimental.pallas import tpu_sc as plsc

_NUM_DOMAINS = 5
_EMB_DIM = 16
_LANES = 16

_NUM_WORKERS = 32
_CHUNK = 512
_IDX_SUB = 128


_DCH = 320


def _make_depad(num_entities):
    steps_total = num_entities // _DCH
    info = plsc.get_sparse_core_info()
    nc = info.num_cores
    mesh = plsc.VectorSubcoreMesh(core_axis_name="c", subcore_axis_name="s")
    kmax = -(-steps_total // _NUM_WORKERS)

    @functools.partial(
        pl.kernel,
        mesh=mesh,
        compiler_params=pltpu.CompilerParams(
            needs_layout_passes=False, use_tc_tiling_on_sc=True),
        out_type=jax.ShapeDtypeStruct((num_entities * _EMB_DIM // 128, 128),
                                      jnp.float32),
        scratch_types=[
            pltpu.VMEM((_DCH, _EMB_DIM), jnp.float32),
            pltpu.VMEM((_DCH * _EMB_DIM // 128, 128), jnp.float32),
        ],
    )
    def depad(in_hbm, out_hbm, buf16_v, buf128_v):
        wid = lax.axis_index("s") * nc + lax.axis_index("c")
        orows = _DCH * _EMB_DIM // 128
        epr = 128 // _EMB_DIM

        def step(k, c1):
            s = wid + _NUM_WORKERS * k

            @pl.when(s < steps_total)
            def _():
                b = pl.multiple_of(s * _DCH, 64)
                pltpu.sync_copy(in_hbm.at[pl.ds(b, _DCH), :], buf16_v)

                def regroup(ro, c2):
                    for e in range(epr):
                        buf128_v[ro, pl.ds(e * _EMB_DIM, _EMB_DIM)] = (
                            buf16_v[ro * epr + e, :])
                    return c2

                lax.fori_loop(0, orows, regroup, 0)
                pltpu.sync_copy(buf128_v,
                                out_hbm.at[pl.ds(
                                    pl.multiple_of(s * orows, 8), orows), :])
            return c1

        lax.fori_loop(0, kmax, step, 0)

    return depad


def _make_sc_scorer(total_samples):
    per_w = total_samples // _NUM_WORKERS
    chunks = per_w // _CHUNK
    idx_per_chunk = _CHUNK * _NUM_DOMAINS
    n_sub = idx_per_chunk // _IDX_SUB

    info = plsc.get_sparse_core_info()
    nc = info.num_cores

    mesh = plsc.VectorSubcoreMesh(core_axis_name="c", subcore_axis_name="s")

    @functools.partial(
        pl.kernel,
        mesh=mesh,
        compiler_params=pltpu.CompilerParams(
            needs_layout_passes=False, use_tc_tiling_on_sc=False),
        out_type=jax.ShapeDtypeStruct((total_samples,), jnp.float32),
        scratch_types=[
            pltpu.VMEM((idx_per_chunk,), jnp.int32),
            pltpu.VMEM((idx_per_chunk, _EMB_DIM), jnp.float32),
            pltpu.VMEM((_CHUNK,), jnp.float32),
            pltpu.VMEM((_LANES,), jnp.float32),
            pltpu.SemaphoreType.DMA,
        ],
    )
    def scorer(table_hbm, idx_hbm, par_hbm, out_hbm,
               idx_v, rows_v, out_v, par_v, sem):
        wid = lax.axis_index("s") * nc + lax.axis_index("c")
        base = wid * per_w

        pltpu.sync_copy(par_hbm, par_v)
        pv = jnp.exp(par_v[:])
        wh = pv[0] * 0.5
        ec = pv[1]
        lanes = lax.iota(jnp.int32, _LANES)

        def chunk_body(g, carry):
            cbase = base + g * _CHUNK
            pltpu.sync_copy(idx_hbm.at[pl.ds(cbase * _NUM_DOMAINS,
                                             idx_per_chunk)], idx_v)

            copies = [
                pltpu.async_copy(
                    table_hbm.at[idx_v.at[pl.ds(k * _IDX_SUB, _IDX_SUB)]],
                    rows_v.at[pl.ds(k * _IDX_SUB, _IDX_SUB)],
                    sem,
                )
                for k in range(n_sub)
            ]
            for cp in copies:
                cp.wait()

            def group_body(j, c2):
                gbase = j * (_LANES * _NUM_DOMAINS)
                vals = jnp.zeros((_LANES,), jnp.float32)
                for i in range(_LANES):
                    s5 = gbase + i * _NUM_DOMAINS
                    e0 = rows_v[s5, :]
                    e1 = rows_v[s5 + 1, :]
                    e2 = rows_v[s5 + 2, :]
                    e3 = rows_v[s5 + 3, :]
                    e4 = rows_v[s5 + 4, :]
                    sv = e0 + e1 + e2 + e3 + e4
                    q = (sv * sv - e0 * e0 - e1 * e1 - e2 * e2
                         - e3 * e3 - e4 * e4)
                    vals = jnp.where(lanes == i, jnp.sum(q), vals)
                out_v[pl.ds(j * _LANES, _LANES)] = jnp.exp(vals * wh) * ec
                return c2

            lax.fori_loop(0, _CHUNK // _LANES, group_body, 0)

            pltpu.sync_copy(out_v, out_hbm.at[pl.ds(cbase, _CHUNK)])
            return carry

        lax.fori_loop(0, chunks, chunk_body, 0)

    return scorer


def kernel(pos_x, neg_x, emb_table, pair_w, c):
    B, N, D = neg_x.shape
    total = B * (1 + N)
    all_idx = jnp.concatenate(
        [pos_x.reshape(-1), neg_x.reshape(-1)]).astype(jnp.int32)
    params = jnp.zeros((_LANES,), jnp.float32)
    params = params.at[0].set(pair_w[0]).at[1].set(c)
    V, E = emb_table.shape
    table_lin = _make_depad(V)(emb_table).reshape(V, E)
    scores = _make_sc_scorer(total)(table_lin, all_idx, params)
    pos_score = scores[:B]
    neg_score = scores[B:].reshape(B, N)
    return pos_score, neg_score


# speedup vs baseline: 1.2980x; 1.2980x over previous
import functools

import jax
import jax.numpy as jnp
from jax import lax
from jax.experimental import pallas as pl
from jax.experimental.pallas import tpu as pltpu
from jax.experimental.pallas import tpu_sc as plsc

_NUM_DOMAINS = 5
_EMB_DIM = 16
_LANES = 16

_NUM_WORKERS = 32
_CHUNK = 512
_IDX_SUB = 128


def _make_sc_scorer(total_samples):
    per_w = total_samples // _NUM_WORKERS
    chunks = per_w // _CHUNK
    idx_per_chunk = _CHUNK * _NUM_DOMAINS
    n_sub = idx_per_chunk // _IDX_SUB

    info = plsc.get_sparse_core_info()
    nc = info.num_cores

    mesh = plsc.VectorSubcoreMesh(core_axis_name="c", subcore_axis_name="s")

    @functools.partial(
        pl.kernel,
        mesh=mesh,
        compiler_params=pltpu.CompilerParams(
            needs_layout_passes=False, use_tc_tiling_on_sc=False),
        out_type=jax.ShapeDtypeStruct((total_samples,), jnp.float32),
        scratch_types=[
            pltpu.VMEM((idx_per_chunk,), jnp.int32),
            pltpu.VMEM((idx_per_chunk,), jnp.int32),
            pltpu.VMEM((idx_per_chunk, _EMB_DIM), jnp.float32),
            pltpu.VMEM((idx_per_chunk, _EMB_DIM), jnp.float32),
            pltpu.VMEM((_CHUNK,), jnp.float32),
            pltpu.VMEM((_LANES,), jnp.float32),
            pltpu.SemaphoreType.DMA,
            pltpu.SemaphoreType.DMA,
        ],
    )
    def scorer(table_hbm, idx_hbm, par_hbm, out_hbm,
               idx0_v, idx1_v, rows0_v, rows1_v, out_v, par_v, sem0, sem1):
        wid = lax.axis_index("s") * nc + lax.axis_index("c")
        base = wid * per_w
        idx_b = (idx0_v, idx1_v)
        rows_b = (rows0_v, rows1_v)
        sem_b = (sem0, sem1)

        pltpu.sync_copy(par_hbm, par_v)
        pv = jnp.exp(par_v[:])
        wh = pv[0] * 0.5
        ec = pv[1]
        lanes = lax.iota(jnp.int32, _LANES)

        def fire(g, par):
            cbase = base + g * _CHUNK
            pltpu.sync_copy(idx_hbm.at[pl.ds(cbase * _NUM_DOMAINS,
                                             idx_per_chunk)], idx_b[par])
            for k in range(n_sub):
                pltpu.async_copy(
                    table_hbm.at[idx_b[par].at[pl.ds(k * _IDX_SUB,
                                                     _IDX_SUB)]],
                    rows_b[par].at[pl.ds(k * _IDX_SUB, _IDX_SUB)],
                    sem_b[par],
                )

        def drain(par):
            pltpu.make_async_copy(
                table_hbm.at[pl.ds(0, idx_per_chunk)],
                rows_b[par], sem_b[par]).wait()

        def compute(g, par):
            rows_v = rows_b[par]

            def group_body(j, c2):
                gbase = j * (_LANES * _NUM_DOMAINS)
                vals = jnp.zeros((_LANES,), jnp.float32)
                for i in range(_LANES):
                    s5 = gbase + i * _NUM_DOMAINS
                    e0 = rows_v[s5, :]
                    e1 = rows_v[s5 + 1, :]
                    e2 = rows_v[s5 + 2, :]
                    e3 = rows_v[s5 + 3, :]
                    e4 = rows_v[s5 + 4, :]
                    sv = e0 + e1 + e2 + e3 + e4
                    q = (sv * sv - e0 * e0 - e1 * e1 - e2 * e2
                         - e3 * e3 - e4 * e4)
                    vals = jnp.where(lanes == i, jnp.sum(q), vals)
                out_v[pl.ds(j * _LANES, _LANES)] = jnp.exp(vals * wh) * ec
                return c2

            lax.fori_loop(0, _CHUNK // _LANES, group_body, 0)
            pltpu.sync_copy(out_v, out_hbm.at[pl.ds(base + g * _CHUNK,
                                                    _CHUNK)])

        fire(0, 0)

        def pair_body(p, carry):
            for par in (0, 1):
                g = p * 2 + par
                drain(par)
                fire(g + 1, 1 - par)
                compute(g, par)
            return carry

        lax.fori_loop(0, (chunks - 1) // 2, pair_body, 0)
        drain(0)
        compute(chunks - 1, 0)

    return scorer


def kernel(pos_x, neg_x, emb_table, pair_w, c):
    B, N, D = neg_x.shape
    total = B * (1 + N)
    all_idx = jnp.concatenate(
        [pos_x.reshape(-1), neg_x.reshape(-1)]).astype(jnp.int32)
    params = jnp.zeros((_LANES,), jnp.float32)
    params = params.at[0].set(pair_w[0]).at[1].set(c)
    scores = _make_sc_scorer(total)(emb_table, all_idx, params)
    pos_score = scores[:B]
    neg_score = scores[B:].reshape(B, N)
    return pos_score, neg_score


# speedup vs baseline: 1.2982x; 1.0001x over previous
import functools

import jax
import jax.numpy as jnp
from jax import lax
from jax.experimental import pallas as pl
from jax.experimental.pallas import tpu as pltpu
from jax.experimental.pallas import tpu_sc as plsc

_NUM_DOMAINS = 5
_EMB_DIM = 16
_LANES = 16

_NUM_WORKERS = 32
_CHUNK = 512
_IDX_SUB = 128


def _make_sc_scorer(total_samples):
    per_w = total_samples // _NUM_WORKERS
    chunks = per_w // _CHUNK
    idx_per_chunk = _CHUNK * _NUM_DOMAINS
    n_sub = idx_per_chunk // _IDX_SUB

    info = plsc.get_sparse_core_info()
    nc = info.num_cores

    mesh = plsc.VectorSubcoreMesh(core_axis_name="c", subcore_axis_name="s")

    @functools.partial(
        pl.kernel,
        mesh=mesh,
        compiler_params=pltpu.CompilerParams(
            needs_layout_passes=False, use_tc_tiling_on_sc=False),
        out_type=jax.ShapeDtypeStruct((total_samples,), jnp.float32),
        scratch_types=[
            pltpu.VMEM((idx_per_chunk,), jnp.int32),
            pltpu.VMEM((idx_per_chunk,), jnp.int32),
            pltpu.VMEM((idx_per_chunk, _EMB_DIM), jnp.float32),
            pltpu.VMEM((idx_per_chunk, _EMB_DIM), jnp.float32),
            pltpu.VMEM((_CHUNK,), jnp.float32),
            pltpu.VMEM((_LANES,), jnp.float32),
            pltpu.SemaphoreType.DMA,
            pltpu.SemaphoreType.DMA,
        ],
    )
    def scorer(table_hbm, idx_hbm, par_hbm, out_hbm,
               idx0_v, idx1_v, rows0_v, rows1_v, out_v, par_v, sem0, sem1):
        wid = lax.axis_index("s") * nc + lax.axis_index("c")
        base = wid * per_w
        idx_b = (idx0_v, idx1_v)
        rows_b = (rows0_v, rows1_v)
        sem_b = (sem0, sem1)

        pltpu.sync_copy(par_hbm, par_v)
        pv = jnp.exp(par_v[:])
        wh = pv[0]
        ec = pv[1]
        lanes = lax.iota(jnp.int32, _LANES)

        def fire(g, par):
            cbase = base + g * _CHUNK
            pltpu.sync_copy(idx_hbm.at[pl.ds(cbase * _NUM_DOMAINS,
                                             idx_per_chunk)], idx_b[par])
            for k in range(n_sub):
                pltpu.async_copy(
                    table_hbm.at[idx_b[par].at[pl.ds(k * _IDX_SUB,
                                                     _IDX_SUB)]],
                    rows_b[par].at[pl.ds(k * _IDX_SUB, _IDX_SUB)],
                    sem_b[par],
                )

        def drain(par):
            pltpu.make_async_copy(
                table_hbm.at[pl.ds(0, idx_per_chunk)],
                rows_b[par], sem_b[par]).wait()

        def compute(g, par):
            rows_v = rows_b[par]

            def group_body(j, c2):
                gbase = j * (_LANES * _NUM_DOMAINS)
                vals = jnp.zeros((_LANES,), jnp.float32)
                for i in range(_LANES):
                    s5 = gbase + i * _NUM_DOMAINS
                    e0 = rows_v[s5, :]
                    e1 = rows_v[s5 + 1, :]
                    e2 = rows_v[s5 + 2, :]
                    e3 = rows_v[s5 + 3, :]
                    e4 = rows_v[s5 + 4, :]
                    s3 = e3 + e4
                    s2 = e2 + s3
                    s1 = e1 + s2
                    q = e0 * s1 + e1 * s2 + e2 * s3 + e3 * e4
                    vals = jnp.where(lanes == i, jnp.sum(q), vals)
                out_v[pl.ds(j * _LANES, _LANES)] = jnp.exp(vals * wh) * ec
                return c2

            lax.fori_loop(0, _CHUNK // _LANES, group_body, 0)
            pltpu.sync_copy(out_v, out_hbm.at[pl.ds(base + g * _CHUNK,
                                                    _CHUNK)])

        fire(0, 0)

        def pair_body(p, carry):
            for par in (0, 1):
                g = p * 2 + par
                drain(par)
                fire(g + 1, 1 - par)
                compute(g, par)
            return carry

        lax.fori_loop(0, (chunks - 1) // 2, pair_body, 0)
        drain(0)
        compute(chunks - 1, 0)

    return scorer


def kernel(pos_x, neg_x, emb_table, pair_w, c):
    B, N, D = neg_x.shape
    total = B * (1 + N)
    all_idx = jnp.concatenate(
        [pos_x.reshape(-1), neg_x.reshape(-1)]).astype(jnp.int32)
    params = jnp.zeros((_LANES,), jnp.float32)
    params = params.at[0].set(pair_w[0]).at[1].set(c)
    scores = _make_sc_scorer(total)(emb_table, all_idx, params)
    pos_score = scores[:B]
    neg_score = scores[B:].reshape(B, N)
    return pos_score, neg_score


# speedup vs baseline: 1.3246x; 1.0203x over previous
import functools

import jax
import jax.numpy as jnp
from jax import lax
from jax.experimental import pallas as pl
from jax.experimental.pallas import tpu as pltpu
from jax.experimental.pallas import tpu_sc as plsc

_NUM_DOMAINS = 5
_EMB_DIM = 16
_LANES = 16

_NUM_WORKERS = 32
_CHUNK = 512
_IDX_SUB = 128


def _make_sc_scorer(total_samples):
    per_w = total_samples // _NUM_WORKERS
    chunks = per_w // _CHUNK
    idx_per_chunk = _CHUNK * _NUM_DOMAINS
    n_sub = idx_per_chunk // _IDX_SUB

    info = plsc.get_sparse_core_info()
    nc = info.num_cores

    mesh = plsc.VectorSubcoreMesh(core_axis_name="c", subcore_axis_name="s")

    @functools.partial(
        pl.kernel,
        mesh=mesh,
        compiler_params=pltpu.CompilerParams(
            needs_layout_passes=False, use_tc_tiling_on_sc=False),
        out_type=jax.ShapeDtypeStruct((total_samples,), jnp.float32),
        scratch_types=[
            pltpu.VMEM((idx_per_chunk,), jnp.int32),
            pltpu.VMEM((idx_per_chunk,), jnp.int32),
            pltpu.VMEM((idx_per_chunk, _EMB_DIM), jnp.float32),
            pltpu.VMEM((idx_per_chunk, _EMB_DIM), jnp.float32),
            pltpu.VMEM((_CHUNK,), jnp.float32),
            pltpu.VMEM((_LANES,), jnp.float32),
            pltpu.SemaphoreType.DMA,
            pltpu.SemaphoreType.DMA,
            pltpu.SemaphoreType.DMA,
            pltpu.SemaphoreType.DMA,
        ],
    )
    def scorer(table_hbm, idx_hbm, par_hbm, out_hbm,
               idx0_v, idx1_v, rows0_v, rows1_v, out_v, par_v,
               sem0, sem1, semi0, semi1):
        wid = lax.axis_index("s") * nc + lax.axis_index("c")
        base = wid * per_w
        idx_b = (idx0_v, idx1_v)
        rows_b = (rows0_v, rows1_v)
        sem_b = (sem0, sem1)
        semi_b = (semi0, semi1)

        pltpu.sync_copy(par_hbm, par_v)
        pv = jnp.exp(par_v[:])
        wh = pv[0]
        ec = pv[1]
        lanes = lax.iota(jnp.int32, _LANES)

        def idx_fetch(g, par):
            cbase = base + g * _CHUNK
            pltpu.async_copy(idx_hbm.at[pl.ds(cbase * _NUM_DOMAINS,
                                              idx_per_chunk)],
                             idx_b[par], semi_b[par])

        def idx_wait(par):
            pltpu.make_async_copy(
                idx_hbm.at[pl.ds(0, idx_per_chunk)],
                idx_b[par], semi_b[par]).wait()

        def fire(g, par):
            for k in range(n_sub):
                pltpu.async_copy(
                    table_hbm.at[idx_b[par].at[pl.ds(k * _IDX_SUB,
                                                     _IDX_SUB)]],
                    rows_b[par].at[pl.ds(k * _IDX_SUB, _IDX_SUB)],
                    sem_b[par],
                )

        def drain(par):
            pltpu.make_async_copy(
                table_hbm.at[pl.ds(0, idx_per_chunk)],
                rows_b[par], sem_b[par]).wait()

        def compute(g, par):
            rows_v = rows_b[par]

            def group_body(j, c2):
                gbase = j * (_LANES * _NUM_DOMAINS)
                vals = jnp.zeros((_LANES,), jnp.float32)
                for i in range(_LANES):
                    s5 = gbase + i * _NUM_DOMAINS
                    e0 = rows_v[s5, :]
                    e1 = rows_v[s5 + 1, :]
                    e2 = rows_v[s5 + 2, :]
                    e3 = rows_v[s5 + 3, :]
                    e4 = rows_v[s5 + 4, :]
                    s3 = e3 + e4
                    s2 = e2 + s3
                    s1 = e1 + s2
                    q = e0 * s1 + e1 * s2 + e2 * s3 + e3 * e4
                    vals = jnp.where(lanes == i, jnp.sum(q), vals)
                out_v[pl.ds(j * _LANES, _LANES)] = jnp.exp(vals * wh) * ec
                return c2

            lax.fori_loop(0, _CHUNK // _LANES, group_body, 0)
            pltpu.sync_copy(out_v, out_hbm.at[pl.ds(base + g * _CHUNK,
                                                    _CHUNK)])

        idx_fetch(0, 0)
        idx_wait(0)
        fire(0, 0)
        idx_fetch(1, 1)

        def pair_body(p, carry):
            for par in (0, 1):
                g = p * 2 + par
                drain(par)
                idx_wait(1 - par)
                fire(g + 1, 1 - par)

                @pl.when(g + 2 < chunks)
                def _():
                    idx_fetch(g + 2, par)

                compute(g, par)
            return carry

        lax.fori_loop(0, (chunks - 1) // 2, pair_body, 0)
        drain(0)
        compute(chunks - 1, 0)

    return scorer


def kernel(pos_x, neg_x, emb_table, pair_w, c):
    B, N, D = neg_x.shape
    total = B * (1 + N)
    all_idx = jnp.concatenate(
        [pos_x.reshape(-1), neg_x.reshape(-1)]).astype(jnp.int32)
    params = jnp.zeros((_LANES,), jnp.float32)
    params = params.at[0].set(pair_w[0]).at[1].set(c)
    scores = _make_sc_scorer(total)(emb_table, all_idx, params)
    pos_score = scores[:B]
    neg_score = scores[B:].reshape(B, N)
    return pos_score, neg_score


# speedup vs baseline: 1.5174x; 1.1456x over previous
import functools

import jax
import jax.numpy as jnp
from jax import lax
from jax.experimental import pallas as pl
from jax.experimental.pallas import tpu as pltpu
from jax.experimental.pallas import tpu_sc as plsc

_NUM_DOMAINS = 5
_EMB_DIM = 16
_LANES = 16

_NUM_WORKERS = 32
_CHUNK = 512
_IDX_SUB = 128


def _make_sc_scorer(total_samples):
    per_w = total_samples // _NUM_WORKERS
    chunks = per_w // _CHUNK
    idx_per_chunk = _CHUNK * _NUM_DOMAINS
    n_sub = idx_per_chunk // _IDX_SUB

    info = plsc.get_sparse_core_info()
    nc = info.num_cores

    mesh = plsc.VectorSubcoreMesh(core_axis_name="c", subcore_axis_name="s")

    @functools.partial(
        pl.kernel,
        mesh=mesh,
        compiler_params=pltpu.CompilerParams(
            needs_layout_passes=False, use_tc_tiling_on_sc=False),
        out_type=jax.ShapeDtypeStruct((total_samples,), jnp.float32),
        scratch_types=[
            pltpu.VMEM((idx_per_chunk,), jnp.int32),
            pltpu.VMEM((idx_per_chunk,), jnp.int32),
            pltpu.VMEM((idx_per_chunk, _EMB_DIM), jnp.float32),
            pltpu.VMEM((idx_per_chunk, _EMB_DIM), jnp.float32),
            pltpu.VMEM((_CHUNK,), jnp.float32),
            pltpu.VMEM((_LANES,), jnp.float32),
            pltpu.SemaphoreType.DMA,
            pltpu.SemaphoreType.DMA,
            pltpu.SemaphoreType.DMA,
            pltpu.SemaphoreType.DMA,
        ],
    )
    def scorer(table_hbm, idx_hbm, par_hbm, out_hbm,
               idx0_v, idx1_v, rows0_v, rows1_v, out_v, par_v,
               sem0, sem1, semi0, semi1):
        wid = lax.axis_index("s") * nc + lax.axis_index("c")
        base = wid * per_w
        idx_b = (idx0_v, idx1_v)
        rows_b = (rows0_v, rows1_v)
        sem_b = (sem0, sem1)
        semi_b = (semi0, semi1)

        pltpu.sync_copy(par_hbm, par_v)
        pv = jnp.exp(par_v[:])
        wh = pv[0]
        ec = pv[1]
        lanes = lax.iota(jnp.int32, _LANES)

        pos_chunks = 16384 // _CHUNK
        pos_n = pos_chunks * _CHUNK
        neg_n = total_samples - pos_n

        def idx_fetch(g, par):
            cg = base // _CHUNK + g
            for d in range(_NUM_DOMAINS):
                off = jnp.where(
                    cg < pos_chunks,
                    d * pos_n + cg * _CHUNK,
                    pos_n * _NUM_DOMAINS + d * neg_n
                    + (cg - pos_chunks) * _CHUNK)
                pltpu.async_copy(
                    idx_hbm.at[pl.ds(off, _CHUNK)],
                    idx_b[par].at[pl.ds(d * _CHUNK, _CHUNK)],
                    semi_b[par])

        def idx_wait(par):
            pltpu.make_async_copy(
                idx_hbm.at[pl.ds(0, idx_per_chunk)],
                idx_b[par], semi_b[par]).wait()

        def fire(g, par):
            for k in range(n_sub):
                pltpu.async_copy(
                    table_hbm.at[idx_b[par].at[pl.ds(k * _IDX_SUB,
                                                     _IDX_SUB)]],
                    rows_b[par].at[pl.ds(k * _IDX_SUB, _IDX_SUB)],
                    sem_b[par],
                )

        def drain(par):
            pltpu.make_async_copy(
                table_hbm.at[pl.ds(0, idx_per_chunk)],
                rows_b[par], sem_b[par]).wait()

        def compute(g, par):
            rows_v = rows_b[par]

            def group_body(j, c2):
                gbase = j * _LANES
                vals = jnp.zeros((_LANES,), jnp.float32)
                for i in range(_LANES):
                    s = gbase + i
                    e0 = rows_v[s, :]
                    e1 = rows_v[s + _CHUNK, :]
                    e2 = rows_v[s + 2 * _CHUNK, :]
                    e3 = rows_v[s + 3 * _CHUNK, :]
                    e4 = rows_v[s + 4 * _CHUNK, :]
                    s3 = e3 + e4
                    s2 = e2 + s3
                    s1 = e1 + s2
                    q = e0 * s1 + e1 * s2 + e2 * s3 + e3 * e4
                    vals = jnp.where(lanes == i, jnp.sum(q), vals)
                out_v[pl.ds(j * _LANES, _LANES)] = jnp.exp(vals * wh) * ec
                return c2

            lax.fori_loop(0, _CHUNK // _LANES, group_body, 0)
            pltpu.sync_copy(out_v, out_hbm.at[pl.ds(base + g * _CHUNK,
                                                    _CHUNK)])

        idx_fetch(0, 0)
        idx_wait(0)
        fire(0, 0)
        idx_fetch(1, 1)

        def pair_body(p, carry):
            for par in (0, 1):
                g = p * 2 + par
                drain(par)
                idx_wait(1 - par)
                fire(g + 1, 1 - par)

                @pl.when(g + 2 < chunks)
                def _():
                    idx_fetch(g + 2, par)

                compute(g, par)
            return carry

        lax.fori_loop(0, (chunks - 1) // 2, pair_body, 0)
        drain(0)
        compute(chunks - 1, 0)

    return scorer


def kernel(pos_x, neg_x, emb_table, pair_w, c):
    B, N, D = neg_x.shape
    total = B * (1 + N)
    all_idx = jnp.concatenate(
        [pos_x.T.reshape(-1),
         jnp.transpose(neg_x, (2, 1, 0)).reshape(-1)]).astype(jnp.int32)
    params = jnp.zeros((_LANES,), jnp.float32)
    params = params.at[0].set(pair_w[0]).at[1].set(c)
    scores = _make_sc_scorer(total)(emb_table, all_idx, params)
    pos_score = scores[:B]
    neg_score = scores[B:].reshape(N, B).T
    return pos_score, neg_score


# speedup vs baseline: 1.5898x; 1.0478x over previous
import functools

import jax
import jax.numpy as jnp
from jax import lax
from jax.experimental import pallas as pl
from jax.experimental.pallas import tpu as pltpu
from jax.experimental.pallas import tpu_sc as plsc

_NUM_DOMAINS = 5
_EMB_DIM = 16
_LANES = 16

_NUM_WORKERS = 32
_CHUNK = 512
_IDX_SUB = 128


_EB = 1024


def _make_sc_transpose(num_entities):
    aligned = (num_entities // 128) * 128
    full_steps = aligned // _EB
    tail = aligned - full_steps * _EB
    info = plsc.get_sparse_core_info()
    nc = info.num_cores
    mesh = plsc.VectorSubcoreMesh(core_axis_name="c", subcore_axis_name="s")
    kmax = -(-(full_steps + 1) // _NUM_WORKERS)

    @functools.partial(
        pl.kernel,
        mesh=mesh,
        compiler_params=pltpu.CompilerParams(
            needs_layout_passes=False, use_tc_tiling_on_sc=True),
        out_type=jax.ShapeDtypeStruct((num_entities * _EMB_DIM // 128, 128),
                                      jnp.float32),
        scratch_types=[
            pltpu.VMEM((_EMB_DIM, _EB), jnp.float32),
            pltpu.VMEM((_EMB_DIM, _EB), jnp.float32),
            pltpu.VMEM((_EB * _EMB_DIM // 128, 128), jnp.float32),
            pltpu.SemaphoreType.DMA,
            pltpu.SemaphoreType.DMA,
        ],
    )
    def transpose_k(in_hbm, out_hbm, in0_v, in1_v, out_v, sem0, sem1):
        wid = lax.axis_index("s") * nc + lax.axis_index("c")
        in_b = (in0_v, in1_v)
        sem_b = (sem0, sem1)
        rows_iota = lax.iota(jnp.int32, _LANES) * _EB

        def fetch(k, par, width):
            s = wid + _NUM_WORKERS * k

            @pl.when(s < full_steps)
            def _():
                eb = pl.multiple_of(s * _EB, 128)
                pltpu.async_copy(in_hbm.at[:, pl.ds(eb, width)],
                                 in_b[par].at[:, pl.ds(0, width)],
                                 sem_b[par])

        def wait_in(par, width):
            pltpu.make_async_copy(
                in_hbm.at[:, pl.ds(0, width)],
                in_b[par].at[:, pl.ds(0, width)], sem_b[par]).wait()

        lanes_i = lax.iota(jnp.int32, _LANES)

        def process(k, par, width):
            s = wid + _NUM_WORKERS * k

            @pl.when(s < full_steps)
            def _():
                wait_in(par, width)
                iv = in_b[par]

                def regroup(e, c2):
                    cols = jnp.broadcast_to(e, (_LANES,))
                    v = plsc.load_gather(iv, [lanes_i, cols])
                    out_v[lax.div(e, 8),
                          pl.ds(lax.rem(e, 8) * _EMB_DIM, _EMB_DIM)] = v
                    return c2

                lax.fori_loop(0, width, regroup, 0)
                orows = width * _EMB_DIM // 128
                pltpu.sync_copy(
                    out_v.at[pl.ds(0, orows), :],
                    out_hbm.at[pl.ds(
                        pl.multiple_of(s * (_EB * _EMB_DIM // 128), 8),
                        orows), :])

        fetch(0, 0, _EB)

        def step(p, c1):
            for par in (0, 1):
                k = p * 2 + par
                fetch(k + 1, 1 - par, _EB)
                process(k, par, _EB)
            return c1

        lax.fori_loop(0, (kmax + 1) // 2, step, 0)

        @pl.when(wid == (0 if tail > 0 else -1))
        def _tail():
            tb = pl.multiple_of(full_steps * _EB, 128)
            pltpu.sync_copy(in_hbm.at[:, pl.ds(tb, tail)],
                            in_b[0].at[:, pl.ds(0, tail)])

            def regroup(e, c2):
                cols = jnp.broadcast_to(e, (_LANES,))
                v = plsc.load_gather(in_b[0], [lanes_i, cols])
                out_v[lax.div(e, 8),
                      pl.ds(lax.rem(e, 8) * _EMB_DIM, _EMB_DIM)] = v
                return c2

            lax.fori_loop(0, tail, regroup, 0)
            orows = tail * _EMB_DIM // 128
            pltpu.sync_copy(
                out_v.at[pl.ds(0, orows), :],
                out_hbm.at[pl.ds(full_steps * (_EB * _EMB_DIM // 128),
                                 orows), :])

    return transpose_k, full_steps, tail


def _make_sc_scorer(total_samples):
    per_w = total_samples // _NUM_WORKERS
    chunks = per_w // _CHUNK
    idx_per_chunk = _CHUNK * _NUM_DOMAINS
    n_sub = idx_per_chunk // _IDX_SUB

    info = plsc.get_sparse_core_info()
    nc = info.num_cores

    mesh = plsc.VectorSubcoreMesh(core_axis_name="c", subcore_axis_name="s")

    @functools.partial(
        pl.kernel,
        mesh=mesh,
        compiler_params=pltpu.CompilerParams(
            needs_layout_passes=False, use_tc_tiling_on_sc=False),
        out_type=jax.ShapeDtypeStruct((total_samples,), jnp.float32),
        scratch_types=[
            pltpu.VMEM((idx_per_chunk,), jnp.int32),
            pltpu.VMEM((idx_per_chunk,), jnp.int32),
            pltpu.VMEM((idx_per_chunk, _EMB_DIM), jnp.float32),
            pltpu.VMEM((idx_per_chunk, _EMB_DIM), jnp.float32),
            pltpu.VMEM((_CHUNK,), jnp.float32),
            pltpu.VMEM((_LANES,), jnp.float32),
            pltpu.SemaphoreType.DMA,
            pltpu.SemaphoreType.DMA,
            pltpu.SemaphoreType.DMA,
            pltpu.SemaphoreType.DMA,
        ],
    )
    def scorer(table_hbm, idx_hbm, par_hbm, out_hbm,
               idx0_v, idx1_v, rows0_v, rows1_v, out_v, par_v,
               sem0, sem1, semi0, semi1):
        wid = lax.axis_index("s") * nc + lax.axis_index("c")
        base = wid * per_w
        idx_b = (idx0_v, idx1_v)
        rows_b = (rows0_v, rows1_v)
        sem_b = (sem0, sem1)
        semi_b = (semi0, semi1)

        pltpu.sync_copy(par_hbm, par_v)
        pv = jnp.exp(par_v[:])
        wh = pv[0]
        ec = pv[1]
        lanes = lax.iota(jnp.int32, _LANES)

        pos_chunks = 16384 // _CHUNK
        pos_n = pos_chunks * _CHUNK
        neg_n = total_samples - pos_n

        def idx_fetch(g, par):
            cg = base // _CHUNK + g
            for d in range(_NUM_DOMAINS):
                off = jnp.where(
                    cg < pos_chunks,
                    d * pos_n + cg * _CHUNK,
                    pos_n * _NUM_DOMAINS + d * neg_n
                    + (cg - pos_chunks) * _CHUNK)
                pltpu.async_copy(
                    idx_hbm.at[pl.ds(off, _CHUNK)],
                    idx_b[par].at[pl.ds(d * _CHUNK, _CHUNK)],
                    semi_b[par])

        def idx_wait(par):
            pltpu.make_async_copy(
                idx_hbm.at[pl.ds(0, idx_per_chunk)],
                idx_b[par], semi_b[par]).wait()

        def fire(g, par):
            for k in range(n_sub):
                pltpu.async_copy(
                    table_hbm.at[idx_b[par].at[pl.ds(k * _IDX_SUB,
                                                     _IDX_SUB)]],
                    rows_b[par].at[pl.ds(k * _IDX_SUB, _IDX_SUB)],
                    sem_b[par],
                )

        def drain(par):
            pltpu.make_async_copy(
                table_hbm.at[pl.ds(0, idx_per_chunk)],
                rows_b[par], sem_b[par]).wait()

        def compute(g, par):
            rows_v = rows_b[par]

            def group_body(j, c2):
                gbase = j * _LANES
                vals = jnp.zeros((_LANES,), jnp.float32)
                for i in range(_LANES):
                    s = gbase + i
                    e0 = rows_v[s, :]
                    e1 = rows_v[s + _CHUNK, :]
                    e2 = rows_v[s + 2 * _CHUNK, :]
                    e3 = rows_v[s + 3 * _CHUNK, :]
                    e4 = rows_v[s + 4 * _CHUNK, :]
                    s3 = e3 + e4
                    s2 = e2 + s3
                    s1 = e1 + s2
                    q = e0 * s1 + e1 * s2 + e2 * s3 + e3 * e4
                    vals = jnp.where(lanes == i, jnp.sum(q), vals)
                out_v[pl.ds(j * _LANES, _LANES)] = jnp.exp(vals * wh) * ec
                return c2

            lax.fori_loop(0, _CHUNK // _LANES, group_body, 0)
            pltpu.sync_copy(out_v, out_hbm.at[pl.ds(base + g * _CHUNK,
                                                    _CHUNK)])

        idx_fetch(0, 0)
        idx_wait(0)
        fire(0, 0)
        idx_fetch(1, 1)

        def pair_body(p, carry):
            for par in (0, 1):
                g = p * 2 + par
                drain(par)
                idx_wait(1 - par)
                fire(g + 1, 1 - par)

                @pl.when(g + 2 < chunks)
                def _():
                    idx_fetch(g + 2, par)

                compute(g, par)
            return carry

        lax.fori_loop(0, (chunks - 1) // 2, pair_body, 0)
        drain(0)
        compute(chunks - 1, 0)

    return scorer


def kernel(pos_x, neg_x, emb_table, pair_w, c):
    B, N, D = neg_x.shape
    total = B * (1 + N)
    all_idx = jnp.concatenate(
        [pos_x.T.reshape(-1),
         jnp.transpose(neg_x, (2, 1, 0)).reshape(-1)]).astype(jnp.int32)
    params = jnp.zeros((_LANES,), jnp.float32)
    params = params.at[0].set(pair_w[0]).at[1].set(c)
    V, E = emb_table.shape
    trans_k, _, _ = _make_sc_transpose(V)
    t128 = trans_k(emb_table.T)
    v_al = (V // 128) * 128
    if v_al < V:
        patch = emb_table[v_al:].reshape((V - v_al) * E // 128, 128)
        t128 = lax.dynamic_update_slice(t128, patch, (v_al * E // 128, 0))
    table_lin = t128.reshape(V, E)
    scores = _make_sc_scorer(total)(table_lin, all_idx, params)
    pos_score = scores[:B]
    neg_score = scores[B:].reshape(N, B).T
    return pos_score, neg_score
